# Initial kernel scaffold; baseline (speedup 1.0000x reference)
import jax
import jax.numpy as jnp
from jax.experimental import pallas as pl


def kernel(a_hat, x, w):
    raise NotImplementedError("write your pallas kernel here")


# speedup vs baseline: 1.5423x; 1.5423x over previous
import jax
import jax.numpy as jnp
from jax.experimental import pallas as pl
from jax.experimental.pallas import tpu as pltpu

_ROW_TILE = 512


def _round_up(x, m):
    return ((x + m - 1) // m) * m


def _pad2d(arr, rows, cols):
    r, c = arr.shape
    if r == rows and c == cols:
        return arr
    return jnp.pad(arr, ((0, rows - r), (0, cols - c)))


def _fused_gcn_kernel(a_ref, x_ref, w_ref, o_ref, xw_ref):
    @pl.when(pl.program_id(1) == 0)
    def _():
        xb = x_ref[...].astype(jnp.bfloat16)
        wb = w_ref[...].astype(jnp.bfloat16)
        xw = jnp.dot(xb, wb, preferred_element_type=jnp.float32)
        xw_ref[...] = xw.astype(jnp.bfloat16)

    a = a_ref[...].astype(jnp.bfloat16)
    acc = jnp.dot(a, xw_ref[...], preferred_element_type=jnp.float32)
    o_ref[...] = jnp.maximum(acc, 0.0)


@jax.jit
def kernel(a_hat, x, w):
    n = a_hat.shape[0]
    c_in = x.shape[1]
    c_out = w.shape[1]

    k_p = _round_up(n, 128)
    cin_p = _round_up(c_in, 128)
    cout_p = _round_up(c_out, 128)
    rows_p = _round_up(n, 2 * _ROW_TILE)
    nb = rows_p // _ROW_TILE // 2

    a_p = _pad2d(a_hat, rows_p, k_p)
    x_p = _pad2d(x, k_p, cin_p)
    w_p = _pad2d(w, cin_p, cout_p)

    out_p = pl.pallas_call(
        _fused_gcn_kernel,
        out_shape=jax.ShapeDtypeStruct((rows_p, cout_p), jnp.float32),
        grid=(2, nb),
        in_specs=[
            pl.BlockSpec((_ROW_TILE, k_p), lambda i, j: (i * nb + j, 0)),
            pl.BlockSpec((k_p, cin_p), lambda i, j: (0, 0)),
            pl.BlockSpec((cin_p, cout_p), lambda i, j: (0, 0)),
        ],
        out_specs=pl.BlockSpec((_ROW_TILE, cout_p), lambda i, j: (i * nb + j, 0)),
        scratch_shapes=[pltpu.VMEM((k_p, cout_p), jnp.bfloat16)],
        compiler_params=pltpu.CompilerParams(
            dimension_semantics=("parallel", "arbitrary")),
    )(a_p, x_p, w_p)

    return out_p[:n, :c_out]
